# Initial kernel scaffold; baseline (speedup 1.0000x reference)
import jax
import jax.numpy as jnp
from jax.experimental import pallas as pl


def kernel(inputs, tok_table, pos_table):
    raise NotImplementedError("write your pallas kernel here")


# speedup vs baseline: 1.4627x; 1.4627x over previous
import functools

import jax
import jax.numpy as jnp
from jax import lax
from jax.experimental import pallas as pl
from jax.experimental.pallas import tpu as pltpu
from jax.experimental.pallas import tpu_sc as plsc

B = 4096
L = 200
D = 32
ROWS = B * L
NW = 32
RPW = ROWS // NW
CHUNK = 1600
NCHUNK = RPW // CHUNK
REPS = CHUNK // L
LANES = 16

_mesh = plsc.VectorSubcoreMesh(core_axis_name="c", subcore_axis_name="s")


@functools.partial(
    pl.kernel,
    mesh=_mesh,
    out_type=jax.ShapeDtypeStruct((ROWS, D), jnp.float32),
    compiler_params=pltpu.CompilerParams(use_tc_tiling_on_sc=False),
    scratch_types=[
        pltpu.VMEM((CHUNK,), jnp.int32),
        pltpu.VMEM((CHUNK,), jnp.int32),
        pltpu.VMEM((CHUNK, D), jnp.float32),
        pltpu.VMEM((CHUNK, D), jnp.float32),
        pltpu.VMEM((L, D), jnp.float32),
        pltpu.SemaphoreType.DMA,
        pltpu.SemaphoreType.DMA,
    ],
)
def _emb_lookup(idx_hbm, tok_hbm, pos_hbm, out_hbm,
                idx0, idx1, rows0, rows1, pos_v, sem0, sem1):
    w = lax.axis_index("s") * 2 + lax.axis_index("c")
    base = w * RPW

    pltpu.sync_copy(pos_hbm, pos_v)

    idx_bufs = (idx0, idx1)
    row_bufs = (rows0, rows1)
    sems = (sem0, sem1)

    for p in range(2):
        off = base + p * CHUNK
        pltpu.sync_copy(idx_hbm.at[pl.ds(off, CHUNK)], idx_bufs[p])
        pltpu.async_copy(tok_hbm.at[idx_bufs[p]], row_bufs[p], sems[p])

    for c in range(NCHUNK):
        bsel = c % 2
        idx_v = idx_bufs[bsel]
        rows = row_bufs[bsel]
        sem = sems[bsel]
        pltpu.make_async_copy(tok_hbm.at[idx_v], rows, sem).wait()

        def add_body(l, _):
            for rep in range(REPS):
                for h in range(D // LANES):
                    sl = pl.ds(h * LANES, LANES)
                    rows[rep * L + l, sl] = rows[rep * L + l, sl] + pos_v[l, sl]
            return 0

        lax.fori_loop(0, L, add_body, 0)

        off = base + c * CHUNK
        pltpu.sync_copy(rows, out_hbm.at[pl.ds(off, CHUNK)])

        nxt = c + 2
        if nxt < NCHUNK:
            noff = base + nxt * CHUNK
            pltpu.sync_copy(idx_hbm.at[pl.ds(noff, CHUNK)], idx_v)
            pltpu.async_copy(tok_hbm.at[idx_v], rows, sem)


def kernel(inputs, tok_table, pos_table):
    idx_flat = inputs.reshape(ROWS).astype(jnp.int32)
    out = _emb_lookup(idx_flat, tok_table, pos_table)
    return out.reshape(B, L, D)
